# baseline (speedup 1.0000x reference)
import functools

import jax
import jax.numpy as jnp
from jax import lax
from jax.experimental import pallas as pl
from jax.experimental.pallas import tpu as pltpu
from jax.experimental.pallas import tpu_sc as plsc

N = 10000
E = 320000
D = 128
NDIV = 9
NHEADS = 2
NB = 400
NBLK = N // NB
NPAD = 10240
RP = NPAD // 16
EP = E // 32
CH = 50
NCH = EP // CH
IB = 40
NIB = NCH // IB
ZR = 16


def _msg_gidx_body(f_ref, n_ref, w_ref, col_ref, rel_ref, o_ref, gi_ref):
    feat = f_ref[...] * n_ref[...]
    for j in range(NDIV):
        o_ref[j] = jnp.dot(feat, w_ref[j], preferred_element_type=jnp.float32)
    gi_ref[...] = rel_ref[...] * N + col_ref[...]


def _msg_body(f_ref, n_ref, w_ref, o_ref):
    feat = f_ref[...] * n_ref[...]
    for j in range(NDIV):
        o_ref[j] = jnp.dot(feat, w_ref[j], preferred_element_type=jnp.float32)


def _sc_body(mtab, gidx, rowi, out,
             g0_v, g1_v, s0_v, s1_v, rows0_v, rows1_v, zero_v, acc,
             sem0, sem1, semi0, semi1):
    c = lax.axis_index("c")
    s = lax.axis_index("s")
    base = (c * 16 + s) * NCH
    gbufs = (g0_v, g1_v)
    sbufs = (s0_v, s1_v)
    isems = (semi0, semi1)

    def idx_load(b, par):
        c0 = pltpu.async_copy(gidx.at[pl.ds(base + b * IB, IB)], gbufs[par],
                              isems[par])
        c1 = pltpu.async_copy(rowi.at[pl.ds(base + b * IB, IB)], sbufs[par],
                              isems[par])
        return c0, c1

    ld = idx_load(0, 0)

    def zrow(i, _):
        for j in range(D // 16):
            zero_v[i, pl.ds(j * 16, 16)] = jnp.zeros((16,), jnp.float32)
        return 0
    lax.fori_loop(0, ZR, zrow, 0)

    def zcopy(k0, _):
        pltpu.sync_copy(zero_v, acc.at[pl.ds(s * RP + k0 * ZR, ZR)])
        return 0
    lax.fori_loop(0, RP // ZR, zcopy, 0)
    plsc.subcore_barrier()

    for b in range(NIB):
        par = b % 2
        gv, sv = gbufs[par], sbufs[par]
        ld[0].wait()
        ld[1].wait()
        if b + 1 < NIB:
            ld = idx_load(b + 1, 1 - par)
        pltpu.async_copy(mtab.at[gv.at[0]], rows0_v, sem0)

        def pair(p, _):
            ci = p * 2
            pltpu.make_async_copy(mtab.at[gv.at[ci]], rows0_v, sem0).wait()
            pltpu.async_copy(mtab.at[gv.at[ci + 1]], rows1_v, sem1)
            pltpu.sync_copy(rows0_v, acc.at[sv.at[ci]], add=True)
            pltpu.make_async_copy(mtab.at[gv.at[ci + 1]], rows1_v, sem1).wait()

            @pl.when(ci + 2 < IB)
            def _():
                pltpu.async_copy(mtab.at[gv.at[ci + 2]], rows0_v, sem0)
            pltpu.sync_copy(rows1_v, acc.at[sv.at[ci + 1]], add=True)
            return 0
        lax.fori_loop(0, IB // 2, pair, 0)
    plsc.subcore_barrier()

    pltpu.sync_copy(acc.at[pl.ds(s * RP, RP)],
                    out.at[pl.ds(c * NPAD + s * RP, RP)])


def _final_body(p0_ref, p1_ref, n_ref, o_ref):
    p0 = p0_ref[...]
    p1 = p1_ref[...]
    nrm = n_ref[...] * (1.0 / NDIV)
    o_ref[...] = 0.5 * (jnp.maximum((p0[0] + p0[1]) * nrm, 0.0)
                        + jnp.maximum((p1[0] + p1[1]) * nrm, 0.0))


def _make_sc_call():
    mesh = plsc.VectorSubcoreMesh(core_axis_name="c", subcore_axis_name="s")
    return pl.kernel(
        _sc_body,
        out_type=jax.ShapeDtypeStruct((2 * NPAD, D), jnp.float32),
        mesh=mesh,
        scratch_types=[
            pltpu.VMEM((IB, CH), jnp.int32),
            pltpu.VMEM((IB, CH), jnp.int32),
            pltpu.VMEM((IB, CH), jnp.int32),
            pltpu.VMEM((IB, CH), jnp.int32),
            pltpu.VMEM((CH, D), jnp.float32),
            pltpu.VMEM((CH, D), jnp.float32),
            pltpu.VMEM((ZR, D), jnp.float32),
            pltpu.VMEM_SHARED((NPAD, D), jnp.float32),
            pltpu.SemaphoreType.DMA,
            pltpu.SemaphoreType.DMA,
            pltpu.SemaphoreType.DMA,
            pltpu.SemaphoreType.DMA,
        ],
    )


def kernel(features, norm, W, edge_index, edge_relation):
    row = edge_index[0].astype(jnp.int32)
    col = edge_index[1].astype(jnp.int32)
    rel = edge_relation.astype(jnp.int32)
    w = W.reshape(NHEADS, NDIV, D, D)

    mtab0, gidx = pl.pallas_call(
        _msg_gidx_body,
        grid=(NBLK,),
        in_specs=[
            pl.BlockSpec((NB, D), lambda i: (i, 0)),
            pl.BlockSpec((NB, 1), lambda i: (i, 0)),
            pl.BlockSpec((NDIV, D, D), lambda i: (0, 0, 0)),
            pl.BlockSpec((E // D, D), lambda i: (0, 0)),
            pl.BlockSpec((E // D, D), lambda i: (0, 0)),
        ],
        out_specs=[
            pl.BlockSpec((NDIV, NB, D), lambda i: (0, i, 0)),
            pl.BlockSpec((E // D, D), lambda i: (0, 0)),
        ],
        out_shape=[
            jax.ShapeDtypeStruct((NDIV, N, D), jnp.float32),
            jax.ShapeDtypeStruct((E // D, D), jnp.int32),
        ],
    )(features, norm, w[0], col.reshape(E // D, D), rel.reshape(E // D, D))

    sc_call = _make_sc_call()
    gidx2 = gidx.reshape(E // CH, CH)
    row2 = row.reshape(E // CH, CH)

    acc0 = sc_call(mtab0.reshape(NDIV * N, D), gidx2, row2)

    mtab1 = pl.pallas_call(
        _msg_body,
        grid=(NBLK,),
        in_specs=[
            pl.BlockSpec((NB, D), lambda i: (i, 0)),
            pl.BlockSpec((NB, 1), lambda i: (i, 0)),
            pl.BlockSpec((NDIV, D, D), lambda i: (0, 0, 0)),
        ],
        out_specs=pl.BlockSpec((NDIV, NB, D), lambda i: (0, i, 0)),
        out_shape=jax.ShapeDtypeStruct((NDIV, N, D), jnp.float32),
    )(features, norm, w[1])

    acc1 = sc_call(mtab1.reshape(NDIV * N, D), gidx2, row2)

    out = pl.pallas_call(
        _final_body,
        grid=(NBLK,),
        in_specs=[
            pl.BlockSpec((2, NB, D), lambda i: (0, i, 0)),
            pl.BlockSpec((2, NB, D), lambda i: (0, i, 0)),
            pl.BlockSpec((NB, 1), lambda i: (i, 0)),
        ],
        out_specs=pl.BlockSpec((NB, D), lambda i: (i, 0)),
        out_shape=jax.ShapeDtypeStruct((N, D), jnp.float32),
    )(acc0.reshape(2, NPAD, D), acc1.reshape(2, NPAD, D), norm)
    return out
